# baseline (speedup 1.0000x reference)
import functools

import jax
import jax.numpy as jnp
from jax import lax
from jax.experimental import pallas as pl
from jax.experimental.pallas import tpu as pltpu
from jax.experimental.pallas import tpu_sc as plsc

_CONF = 0.3
_NCLS = 91
_NW = 32
_ROWS = 57600
_RPW = _ROWS // _NW
_C = 360
_NCHUNK = _RPW // _C
_NGRP = (_C + 15) // 16


def _body(logits_hbm, boxes_hbm, scale_hbm, det_hbm, mask_hbm,
          lbuf0, lbuf1, bbuf, dbuf, mbuf, sbuf, sem0, sem1, semb):
    wid = lax.axis_index("c") * 16 + lax.axis_index("s")
    base_row = wid * _RPW

    pltpu.sync_copy(scale_hbm, sbuf)
    wv = sbuf[pl.ds(0, 16)]
    hv = sbuf[pl.ds(16, 16)]
    iota = lax.iota(jnp.int32, 16)

    lbufs = (lbuf0, lbuf1)
    sems = (sem0, sem1)

    def start_load(k, buf, sem):
        row0 = base_row + k * _C
        return pltpu.async_copy(
            logits_hbm.at[pl.ds(row0 * _NCLS, _C * _NCLS)], buf, sem)

    cp = start_load(0, lbufs[0], sems[0])

    for k in range(_NCHUNK):
        row0 = base_row + k * _C
        lbuf = lbufs[k % 2]
        cp.wait()
        if k + 1 < _NCHUNK:
            cp = start_load(k + 1, lbufs[(k + 1) % 2], sems[(k + 1) % 2])
        pltpu.async_copy(
            boxes_hbm.at[pl.ds(row0 * 4, _C * 4)], bbuf, semb).wait()

        def group(g, _):
            base = jnp.minimum(g * 16, _C - 16)
            rowv = iota + base
            lidx = rowv * _NCLS
            m = plsc.load_gather(lbuf, [lidx])
            am = jnp.zeros((16,), jnp.float32)
            for cc in range(1, _NCLS):
                lidx = lidx + 1
                v = plsc.load_gather(lbuf, [lidx])
                gt = v > m
                m = jnp.where(gt, v, m)
                am = jnp.where(gt, float(cc), am)
            s = 1.0 / (1.0 + jnp.exp(-m))
            keep = s >= _CONF
            zero = jnp.zeros((16,), jnp.float32)

            bidx = rowv * 4
            cx = plsc.load_gather(bbuf, [bidx])
            cy = plsc.load_gather(bbuf, [bidx + 1])
            bw = plsc.load_gather(bbuf, [bidx + 2])
            bh = plsc.load_gather(bbuf, [bidx + 3])
            x = (cx - 0.5 * bw) * wv
            y = (cy - 0.5 * bh) * hv
            ow = bw * wv
            oh = bh * hv

            didx = rowv * 6
            plsc.store_scatter(dbuf, [didx], jnp.where(keep, am, zero))
            plsc.store_scatter(dbuf, [didx + 1], jnp.where(keep, s, zero))
            plsc.store_scatter(dbuf, [didx + 2], jnp.where(keep, x, zero))
            plsc.store_scatter(dbuf, [didx + 3], jnp.where(keep, y, zero))
            plsc.store_scatter(dbuf, [didx + 4], jnp.where(keep, ow, zero))
            plsc.store_scatter(dbuf, [didx + 5], jnp.where(keep, oh, zero))
            plsc.store_scatter(mbuf, [rowv], keep.astype(jnp.int32))
            return 0

        lax.fori_loop(0, _NGRP, group, 0)

        pltpu.async_copy(
            dbuf, det_hbm.at[pl.ds(row0 * 6, _C * 6)], semb).wait()
        pltpu.async_copy(
            mbuf, mask_hbm.at[pl.ds(row0, _C)], semb).wait()


@functools.partial(jax.jit, static_argnames=())
def _run(logits_flat, boxes_flat, scale):
    fn = pl.kernel(
        _body,
        out_type=(
            jax.ShapeDtypeStruct((_ROWS * 6,), jnp.float32),
            jax.ShapeDtypeStruct((_ROWS,), jnp.int32),
        ),
        mesh=plsc.VectorSubcoreMesh(core_axis_name="c", subcore_axis_name="s"),
        scratch_types=[
            pltpu.VMEM((_C * _NCLS,), jnp.float32),
            pltpu.VMEM((_C * _NCLS,), jnp.float32),
            pltpu.VMEM((_C * 4,), jnp.float32),
            pltpu.VMEM((_C * 6,), jnp.float32),
            pltpu.VMEM((_C,), jnp.int32),
            pltpu.VMEM((32,), jnp.float32),
            pltpu.SemaphoreType.DMA,
            pltpu.SemaphoreType.DMA,
            pltpu.SemaphoreType.DMA,
        ],
        compiler_params=pltpu.CompilerParams(needs_layout_passes=False),
    )
    return fn(logits_flat, boxes_flat, scale)


def kernel(logits, boxes, img_h, img_w):
    n, q, c = logits.shape
    logits_flat = logits.reshape(n * q * c)
    boxes_flat = boxes.reshape(n * q * 4)
    fw = jnp.full((16,), img_w, jnp.float32)
    fh = jnp.full((16,), img_h, jnp.float32)
    scale = jnp.concatenate([fw, fh])
    det_flat, mask_i32 = _run(logits_flat, boxes_flat, scale)
    det = det_flat.reshape(n, q, 6)
    mask = mask_i32.astype(bool).reshape(n, q)
    return det, mask


# speedup vs baseline: 6.0057x; 6.0057x over previous
import functools

import jax
import jax.numpy as jnp
from jax import lax
from jax.experimental import pallas as pl
from jax.experimental.pallas import tpu as pltpu
from jax.experimental.pallas import tpu_sc as plsc

_CONF = 0.3
_NCLS = 91
_NIMG = 64
_NQ = 900
_QSPLITS = ((0, 23), (23, 23), (46, 23), (69, 22))
_NFULL = 7
_FULL_WORKERS = 28
_QTAIL = _NFULL * 128


def _body(lg, bx, sc, tl, tb, det, tdet,
          qb0, qb1, qb2, qb3, bb0, bb1, bb2, bb3,
          db0, db1, db2, db3, db4, db5, mb, ab, sbuf,
          tlb, tbb, tdb):
    qbs = (qb0, qb1, qb2, qb3)
    bbs = (bb0, bb1, bb2, bb3)
    dbs = (db0, db1, db2, db3, db4, db5)

    wid = lax.axis_index("c") * 16 + lax.axis_index("s")
    pltpu.sync_copy(sc, sbuf)
    wv = sbuf[pl.ds(0, 16)]
    hv = sbuf[pl.ds(16, 16)]
    is_tail = wid >= _FULL_WORKERS

    @pl.when(is_tail)
    def _():
        tw = wid - _FULL_WORKERS
        r0 = tw * 64
        pltpu.sync_copy(tl.at[pl.ds(r0 * _NCLS, 64 * _NCLS)], tlb)
        pltpu.sync_copy(tb.at[pl.ds(r0 * 4, 64 * 4)], tbb)
        iota = lax.iota(jnp.int32, 16)

        def tgroup(g, _):
            rowv = iota + g * 16
            lidx = rowv * _NCLS
            m = plsc.load_gather(tlb, [lidx])
            am = jnp.zeros((16,), jnp.float32)
            for cc in range(1, _NCLS):
                lidx = lidx + 1
                v = plsc.load_gather(tlb, [lidx])
                gt = v > m
                m = jnp.where(gt, v, m)
                am = jnp.where(gt, float(cc), am)
            s = 1.0 / (1.0 + jnp.exp(-m))
            keep = s >= _CONF
            zero = jnp.zeros((16,), jnp.float32)
            bidx = rowv * 4
            cx = plsc.load_gather(tbb, [bidx])
            cy = plsc.load_gather(tbb, [bidx + 1])
            bw = plsc.load_gather(tbb, [bidx + 2])
            bh = plsc.load_gather(tbb, [bidx + 3])
            didx = rowv * 6
            plsc.store_scatter(tdb, [didx], jnp.where(keep, am, zero))
            plsc.store_scatter(tdb, [didx + 1], jnp.where(keep, s, zero))
            plsc.store_scatter(
                tdb, [didx + 2], jnp.where(keep, (cx - 0.5 * bw) * wv, zero))
            plsc.store_scatter(
                tdb, [didx + 3], jnp.where(keep, (cy - 0.5 * bh) * hv, zero))
            plsc.store_scatter(tdb, [didx + 4], jnp.where(keep, bw * wv, zero))
            plsc.store_scatter(tdb, [didx + 5], jnp.where(keep, bh * hv, zero))
            return 0

        lax.fori_loop(0, 4, tgroup, 0)
        pltpu.sync_copy(tdb, tdet.at[pl.ds(r0 * 6, 64 * 6)])

    def position(t, _):
        p_full = 2 * wid + t
        band = p_full // _NFULL
        k = p_full % _NFULL
        n0 = pl.multiple_of(band * 8, 8)
        q0 = pl.multiple_of(k * 128, 128)

        for (c0, cn), qb in zip(_QSPLITS, qbs):
            pltpu.sync_copy(
                lg.at[pl.ds(c0, cn), pl.ds(n0, 8), pl.ds(q0, 128)],
                qb.at[pl.ds(0, cn)])
        for comp in range(4):
            pltpu.sync_copy(
                bx.at[pl.ds(n0, 8), comp, pl.ds(q0, 128)], bbs[comp])

        for qi, ((c0, cn), qb) in enumerate(zip(_QSPLITS, qbs)):
            for i in range(8):
                def jbody(j, _, qi=qi, c0=c0, cn=cn, qb=qb, i=i):
                    off = pl.multiple_of(j * 16, 16)
                    if qi == 0:
                        m = qb[0, i, pl.ds(off, 16)]
                        am = jnp.zeros((16,), jnp.float32)
                        cstart = 1
                    else:
                        m = mb[i, pl.ds(off, 16)]
                        am = ab[i, pl.ds(off, 16)]
                        cstart = 0
                    for cl in range(cstart, cn):
                        v = qb[cl, i, pl.ds(off, 16)]
                        gt = v > m
                        m = jnp.where(gt, v, m)
                        am = jnp.where(gt, float(c0 + cl), am)
                    if qi < 3:
                        mb[i, pl.ds(off, 16)] = m
                        ab[i, pl.ds(off, 16)] = am
                    else:
                        s = 1.0 / (1.0 + jnp.exp(-m))
                        keep = s >= _CONF
                        zero = jnp.zeros((16,), jnp.float32)
                        cx = bb0[i, pl.ds(off, 16)]
                        cy = bb1[i, pl.ds(off, 16)]
                        bw = bb2[i, pl.ds(off, 16)]
                        bh = bb3[i, pl.ds(off, 16)]
                        db0[i, pl.ds(off, 16)] = jnp.where(keep, am, zero)
                        db1[i, pl.ds(off, 16)] = jnp.where(keep, s, zero)
                        db2[i, pl.ds(off, 16)] = jnp.where(
                            keep, (cx - 0.5 * bw) * wv, zero)
                        db3[i, pl.ds(off, 16)] = jnp.where(
                            keep, (cy - 0.5 * bh) * hv, zero)
                        db4[i, pl.ds(off, 16)] = jnp.where(keep, bw * wv, zero)
                        db5[i, pl.ds(off, 16)] = jnp.where(keep, bh * hv, zero)
                    return 0
                lax.fori_loop(0, 8, jbody, 0)

        for comp in range(6):
            pltpu.sync_copy(
                dbs[comp],
                det.at[comp, pl.ds(n0, 8), pl.ds(q0, 128)])
        return 0

    @pl.when(jnp.logical_not(is_tail))
    def _():
        lax.fori_loop(0, 2, position, 0)


@jax.jit
def _run(lg_t, bx_t, scale, tl, tb):
    fn = pl.kernel(
        _body,
        out_type=(
            jax.ShapeDtypeStruct((6, _NIMG, _NQ), jnp.float32),
            jax.ShapeDtypeStruct((_NIMG * 4 * 6,), jnp.float32),
        ),
        mesh=plsc.VectorSubcoreMesh(core_axis_name="c", subcore_axis_name="s"),
        scratch_types=(
            [pltpu.VMEM((23, 8, 128), jnp.float32) for _ in range(4)]
            + [pltpu.VMEM((8, 128), jnp.float32) for _ in range(4)]
            + [pltpu.VMEM((8, 128), jnp.float32) for _ in range(6)]
            + [pltpu.VMEM((8, 128), jnp.float32) for _ in range(2)]
            + [pltpu.VMEM((32,), jnp.float32)]
            + [pltpu.VMEM((64 * _NCLS,), jnp.float32),
               pltpu.VMEM((64 * 4,), jnp.float32),
               pltpu.VMEM((64 * 6,), jnp.float32)]
        ),
        compiler_params=pltpu.CompilerParams(
            needs_layout_passes=False, use_tc_tiling_on_sc=True),
    )
    return fn(lg_t, bx_t, scale, tl, tb)


def kernel(logits, boxes, img_h, img_w):
    n, q, c = logits.shape
    lg_t = jnp.transpose(logits, (2, 0, 1))
    bx_t = jnp.transpose(boxes, (0, 2, 1))
    fw = jnp.full((16,), img_w, jnp.float32)
    fh = jnp.full((16,), img_h, jnp.float32)
    scale = jnp.concatenate([fw, fh])
    tl = logits[:, _QTAIL:, :].reshape(-1)
    tb = boxes[:, _QTAIL:, :].reshape(-1)
    det_t, tdet = _run(lg_t, bx_t, scale, tl, tb)
    det = jnp.transpose(det_t, (1, 2, 0))
    det = det.at[:, _QTAIL:, :].set(tdet.reshape(n, q - _QTAIL, 6))
    mask = det[..., 1] >= _CONF
    return det, mask


# speedup vs baseline: 7.2228x; 1.2027x over previous
import functools

import jax
import jax.numpy as jnp
from jax import lax
from jax.experimental import pallas as pl
from jax.experimental.pallas import tpu as pltpu
from jax.experimental.pallas import tpu_sc as plsc

_CONF = 0.3
_NCLS = 91
_NIMG = 64
_NQ = 900
_QSPLITS = ((0, 23), (23, 23), (46, 23), (69, 22))
_NFULL = 7
_FULL_WORKERS = 28
_QTAIL = _NFULL * 128


def _body(lg, bx, sc, tl, tb, det, tdet,
          qb0, qb1, qb2, qb3, bb0, bb1, bb2, bb3,
          db0, db1, db2, db3, db4, db5, mb, ab, sbuf,
          tlb, tbb, tdb,
          qsem0, qsem1, qsem2, qsem3, bsem):
    qsems = (qsem0, qsem1, qsem2, qsem3)
    qbs = (qb0, qb1, qb2, qb3)
    bbs = (bb0, bb1, bb2, bb3)
    dbs = (db0, db1, db2, db3, db4, db5)

    wid = lax.axis_index("c") * 16 + lax.axis_index("s")
    pltpu.sync_copy(sc, sbuf)
    wv = sbuf[pl.ds(0, 16)]
    hv = sbuf[pl.ds(16, 16)]
    is_tail = wid >= _FULL_WORKERS

    @pl.when(is_tail)
    def _():
        tw = wid - _FULL_WORKERS
        r0 = tw * 64
        pltpu.sync_copy(tl.at[pl.ds(r0 * _NCLS, 64 * _NCLS)], tlb)
        pltpu.sync_copy(tb.at[pl.ds(r0 * 4, 64 * 4)], tbb)
        iota = lax.iota(jnp.int32, 16)

        def tgroup(g, _):
            rowv = iota + g * 16
            lidx = rowv * _NCLS
            m = plsc.load_gather(tlb, [lidx])
            am = jnp.zeros((16,), jnp.float32)
            for cc in range(1, _NCLS):
                lidx = lidx + 1
                v = plsc.load_gather(tlb, [lidx])
                gt = v > m
                m = jnp.where(gt, v, m)
                am = jnp.where(gt, float(cc), am)
            s = 1.0 / (1.0 + jnp.exp(-m))
            keep = s >= _CONF
            zero = jnp.zeros((16,), jnp.float32)
            bidx = rowv * 4
            cx = plsc.load_gather(tbb, [bidx])
            cy = plsc.load_gather(tbb, [bidx + 1])
            bw = plsc.load_gather(tbb, [bidx + 2])
            bh = plsc.load_gather(tbb, [bidx + 3])
            didx = rowv * 6
            plsc.store_scatter(tdb, [didx], jnp.where(keep, am, zero))
            plsc.store_scatter(tdb, [didx + 1], jnp.where(keep, s, zero))
            plsc.store_scatter(
                tdb, [didx + 2], jnp.where(keep, (cx - 0.5 * bw) * wv, zero))
            plsc.store_scatter(
                tdb, [didx + 3], jnp.where(keep, (cy - 0.5 * bh) * hv, zero))
            plsc.store_scatter(tdb, [didx + 4], jnp.where(keep, bw * wv, zero))
            plsc.store_scatter(tdb, [didx + 5], jnp.where(keep, bh * hv, zero))
            return 0

        lax.fori_loop(0, 4, tgroup, 0)
        pltpu.sync_copy(tdb, tdet.at[pl.ds(r0 * 6, 64 * 6)])

    def position(t, _):
        p_full = 2 * wid + t
        band = p_full // _NFULL
        k = p_full % _NFULL
        n0 = pl.multiple_of(band * 8, 8)
        q0 = pl.multiple_of(k * 128, 128)

        cps = [
            pltpu.async_copy(
                lg.at[pl.ds(c0, cn), pl.ds(n0, 8), pl.ds(q0, 128)],
                qb.at[pl.ds(0, cn)], sem)
            for ((c0, cn), qb), sem in zip(zip(_QSPLITS, qbs), qsems)
        ]
        bcps = [
            pltpu.async_copy(
                bx.at[pl.ds(n0, 8), comp, pl.ds(q0, 128)], bbs[comp], bsem)
            for comp in range(4)
        ]

        for qi, ((c0, cn), qb) in enumerate(zip(_QSPLITS, qbs)):
            cps[qi].wait()
            if qi == 3:
                for cp in bcps:
                    cp.wait()

            def ibody(i, _, qi=qi, c0=c0, cn=cn, qb=qb):
                ms, ams = [], []
                for g in range(8):
                    off = pl.multiple_of(g * 16, 16)
                    if qi == 0:
                        m = qb[0, i, pl.ds(off, 16)]
                        am = jnp.zeros((16,), jnp.float32)
                    else:
                        m = mb[i, pl.ds(off, 16)]
                        am = ab[i, pl.ds(off, 16)]
                    ms.append(m)
                    ams.append(am)
                cstart = 1 if qi == 0 else 0
                for cl in range(cstart, cn):
                    for g in range(8):
                        off = pl.multiple_of(g * 16, 16)
                        v = qb[cl, i, pl.ds(off, 16)]
                        gt = v > ms[g]
                        ms[g] = jnp.maximum(ms[g], v)
                        ams[g] = jnp.where(gt, float(c0 + cl), ams[g])
                if qi < 3:
                    for g in range(8):
                        off = pl.multiple_of(g * 16, 16)
                        mb[i, pl.ds(off, 16)] = ms[g]
                        ab[i, pl.ds(off, 16)] = ams[g]
                else:
                    zero = jnp.zeros((16,), jnp.float32)
                    for g in range(8):
                        off = pl.multiple_of(g * 16, 16)
                        m, am = ms[g], ams[g]
                        s = 1.0 / (1.0 + jnp.exp(-m))
                        keep = s >= _CONF
                        cx = bb0[i, pl.ds(off, 16)]
                        cy = bb1[i, pl.ds(off, 16)]
                        bw = bb2[i, pl.ds(off, 16)]
                        bh = bb3[i, pl.ds(off, 16)]
                        db0[i, pl.ds(off, 16)] = jnp.where(keep, am, zero)
                        db1[i, pl.ds(off, 16)] = jnp.where(keep, s, zero)
                        db2[i, pl.ds(off, 16)] = jnp.where(
                            keep, (cx - 0.5 * bw) * wv, zero)
                        db3[i, pl.ds(off, 16)] = jnp.where(
                            keep, (cy - 0.5 * bh) * hv, zero)
                        db4[i, pl.ds(off, 16)] = jnp.where(keep, bw * wv, zero)
                        db5[i, pl.ds(off, 16)] = jnp.where(keep, bh * hv, zero)
                return 0

            lax.fori_loop(0, 8, ibody, 0)

        for comp in range(6):
            pltpu.sync_copy(
                dbs[comp],
                det.at[comp, pl.ds(n0, 8), pl.ds(q0, 128)])
        return 0

    @pl.when(jnp.logical_not(is_tail))
    def _():
        lax.fori_loop(0, 2, position, 0)


@jax.jit
def _run(lg_t, bx_t, scale, tl, tb):
    fn = pl.kernel(
        _body,
        out_type=(
            jax.ShapeDtypeStruct((6, _NIMG, _NQ), jnp.float32),
            jax.ShapeDtypeStruct((_NIMG * 4 * 6,), jnp.float32),
        ),
        mesh=plsc.VectorSubcoreMesh(core_axis_name="c", subcore_axis_name="s"),
        scratch_types=(
            [pltpu.VMEM((23, 8, 128), jnp.float32) for _ in range(4)]
            + [pltpu.VMEM((8, 128), jnp.float32) for _ in range(4)]
            + [pltpu.VMEM((8, 128), jnp.float32) for _ in range(6)]
            + [pltpu.VMEM((8, 128), jnp.float32) for _ in range(2)]
            + [pltpu.VMEM((32,), jnp.float32)]
            + [pltpu.VMEM((64 * _NCLS,), jnp.float32),
               pltpu.VMEM((64 * 4,), jnp.float32),
               pltpu.VMEM((64 * 6,), jnp.float32)]
            + [pltpu.SemaphoreType.DMA for _ in range(5)]
        ),
        compiler_params=pltpu.CompilerParams(
            needs_layout_passes=False, use_tc_tiling_on_sc=True),
    )
    return fn(lg_t, bx_t, scale, tl, tb)


def kernel(logits, boxes, img_h, img_w):
    n, q, c = logits.shape
    lg_t = jnp.transpose(logits, (2, 0, 1))
    bx_t = jnp.transpose(boxes, (0, 2, 1))
    fw = jnp.full((16,), img_w, jnp.float32)
    fh = jnp.full((16,), img_h, jnp.float32)
    scale = jnp.concatenate([fw, fh])
    tl = logits[:, _QTAIL:, :].reshape(-1)
    tb = boxes[:, _QTAIL:, :].reshape(-1)
    det_t, tdet = _run(lg_t, bx_t, scale, tl, tb)
    det = jnp.transpose(det_t, (1, 2, 0))
    det = det.at[:, _QTAIL:, :].set(tdet.reshape(n, q - _QTAIL, 6))
    mask = det[..., 1] >= _CONF
    return det, mask


# speedup vs baseline: 8.9262x; 1.2358x over previous
import jax
import jax.numpy as jnp
from jax import lax
from jax.experimental import pallas as pl
from jax.experimental.pallas import tpu as pltpu
from jax.experimental.pallas import tpu_sc as plsc

_CONF = 0.3
_NCLS = 91
_NIMG = 64
_NQ = 900
_QTAIL = 896
_CCH = 13
_NCH = 7
_W = 256


def _body(lg, bx, sc, tl, tb, det, tdet,
          rb0, rb1, rb2, bb0, bb1, bb2, bb3,
          db0, db1, db2, db3, db4, db5, mb, ab, sbuf,
          tlb, tbb, tdb,
          sem0, sem1, sem2, bsem, tsem):
    rbs = (rb0, rb1, rb2)
    sems = (sem0, sem1, sem2)
    bbs = (bb0, bb1, bb2, bb3)
    dbs = (db0, db1, db2, db3, db4, db5)

    wid = lax.axis_index("c") * 16 + lax.axis_index("s")
    band = wid // 4
    wslot = wid % 4
    n0 = pl.multiple_of(band * 8, 8)
    q0 = pl.multiple_of(jnp.minimum(wslot * _W, _NQ - 4 - _W), 128)
    is_tail = wslot == 3

    pltpu.sync_copy(sc, sbuf)
    wv = sbuf[pl.ds(0, 16)]
    hv = sbuf[pl.ds(16, 16)]

    r0 = band * 32

    @pl.when(is_tail)
    def _():
        pltpu.async_copy(tl.at[pl.ds(r0 * _NCLS, 32 * _NCLS)], tlb, tsem).wait()
        pltpu.async_copy(tb.at[pl.ds(r0 * 4, 32 * 4)], tbb, tsem).wait()

    def start(c):
        return pltpu.async_copy(
            lg.at[pl.ds(c * _CCH, _CCH), pl.ds(n0, 8), pl.ds(q0, _W)],
            rbs[c % 3], sems[c % 3])

    cps = {c: start(c) for c in range(3)}
    bcps = [
        pltpu.async_copy(
            bx.at[pl.ds(n0, 8), comp, pl.ds(q0, _W)], bbs[comp], bsem)
        for comp in range(4)
    ]

    for c in range(_NCH):
        cps[c].wait()
        qb = rbs[c % 3]
        if c == _NCH - 1:
            for cp in bcps:
                cp.wait()

        def ibody(i, _, c=c, qb=qb):
            ms, ams = [], []
            for g in range(16):
                off = pl.multiple_of(g * 16, 16)
                if c == 0:
                    m = qb[0, i, pl.ds(off, 16)]
                    am = jnp.zeros((16,), jnp.float32)
                else:
                    m = mb[i, pl.ds(off, 16)]
                    am = ab[i, pl.ds(off, 16)]
                ms.append(m)
                ams.append(am)
            for cl in range(1 if c == 0 else 0, _CCH):
                for g in range(16):
                    off = pl.multiple_of(g * 16, 16)
                    v = qb[cl, i, pl.ds(off, 16)]
                    gt = v > ms[g]
                    ms[g] = jnp.maximum(ms[g], v)
                    ams[g] = jnp.where(gt, float(c * _CCH + cl), ams[g])
            if c < _NCH - 1:
                for g in range(16):
                    off = pl.multiple_of(g * 16, 16)
                    mb[i, pl.ds(off, 16)] = ms[g]
                    ab[i, pl.ds(off, 16)] = ams[g]
            else:
                zero = jnp.zeros((16,), jnp.float32)
                for g in range(16):
                    off = pl.multiple_of(g * 16, 16)
                    m, am = ms[g], ams[g]
                    s = 1.0 / (1.0 + jnp.exp(-m))
                    keep = s >= _CONF
                    cx = bb0[i, pl.ds(off, 16)]
                    cy = bb1[i, pl.ds(off, 16)]
                    bw = bb2[i, pl.ds(off, 16)]
                    bh = bb3[i, pl.ds(off, 16)]
                    db0[i, pl.ds(off, 16)] = jnp.where(keep, am, zero)
                    db1[i, pl.ds(off, 16)] = jnp.where(keep, s, zero)
                    db2[i, pl.ds(off, 16)] = jnp.where(
                        keep, (cx - 0.5 * bw) * wv, zero)
                    db3[i, pl.ds(off, 16)] = jnp.where(
                        keep, (cy - 0.5 * bh) * hv, zero)
                    db4[i, pl.ds(off, 16)] = jnp.where(keep, bw * wv, zero)
                    db5[i, pl.ds(off, 16)] = jnp.where(keep, bh * hv, zero)
            return 0

        lax.fori_loop(0, 8, ibody, 0)
        if c + 3 < _NCH:
            cps[c + 3] = start(c + 3)

    for comp in range(6):
        pltpu.sync_copy(
            dbs[comp], det.at[comp, pl.ds(n0, 8), pl.ds(q0, _W)])

    @pl.when(is_tail)
    def _():
        iota = lax.iota(jnp.int32, 16)

        def tgroup(g, _):
            rowv = iota + g * 16
            lidx = rowv * _NCLS
            m = plsc.load_gather(tlb, [lidx])
            am = jnp.zeros((16,), jnp.float32)
            for cc in range(1, _NCLS):
                lidx = lidx + 1
                v = plsc.load_gather(tlb, [lidx])
                gt = v > m
                m = jnp.where(gt, v, m)
                am = jnp.where(gt, float(cc), am)
            s = 1.0 / (1.0 + jnp.exp(-m))
            keep = s >= _CONF
            zero = jnp.zeros((16,), jnp.float32)
            bidx = rowv * 4
            cx = plsc.load_gather(tbb, [bidx])
            cy = plsc.load_gather(tbb, [bidx + 1])
            bw = plsc.load_gather(tbb, [bidx + 2])
            bh = plsc.load_gather(tbb, [bidx + 3])
            didx = rowv * 6
            plsc.store_scatter(tdb, [didx], jnp.where(keep, am, zero))
            plsc.store_scatter(tdb, [didx + 1], jnp.where(keep, s, zero))
            plsc.store_scatter(
                tdb, [didx + 2], jnp.where(keep, (cx - 0.5 * bw) * wv, zero))
            plsc.store_scatter(
                tdb, [didx + 3], jnp.where(keep, (cy - 0.5 * bh) * hv, zero))
            plsc.store_scatter(tdb, [didx + 4], jnp.where(keep, bw * wv, zero))
            plsc.store_scatter(tdb, [didx + 5], jnp.where(keep, bh * hv, zero))
            return 0

        lax.fori_loop(0, 2, tgroup, 0)
        pltpu.sync_copy(tdb, tdet.at[pl.ds(r0 * 6, 32 * 6)])


@jax.jit
def _run(lg_t, bx_t, scale, tl, tb):
    fn = pl.kernel(
        _body,
        out_type=(
            jax.ShapeDtypeStruct((6, _NIMG, _NQ), jnp.float32),
            jax.ShapeDtypeStruct((_NIMG * 4 * 6,), jnp.float32),
        ),
        mesh=plsc.VectorSubcoreMesh(core_axis_name="c", subcore_axis_name="s"),
        scratch_types=(
            [pltpu.VMEM((_CCH, 8, _W), jnp.float32) for _ in range(3)]
            + [pltpu.VMEM((8, _W), jnp.float32) for _ in range(4)]
            + [pltpu.VMEM((8, _W), jnp.float32) for _ in range(6)]
            + [pltpu.VMEM((8, _W), jnp.float32) for _ in range(2)]
            + [pltpu.VMEM((32,), jnp.float32)]
            + [pltpu.VMEM((32 * _NCLS,), jnp.float32),
               pltpu.VMEM((32 * 4,), jnp.float32),
               pltpu.VMEM((32 * 6,), jnp.float32)]
            + [pltpu.SemaphoreType.DMA for _ in range(5)]
        ),
        compiler_params=pltpu.CompilerParams(
            needs_layout_passes=False, use_tc_tiling_on_sc=True),
    )
    return fn(lg_t, bx_t, scale, tl, tb)


def kernel(logits, boxes, img_h, img_w):
    n, q, c = logits.shape
    lg_t = jnp.transpose(logits, (2, 0, 1))
    bx_t = jnp.transpose(boxes, (0, 2, 1))
    fw = jnp.full((16,), img_w, jnp.float32)
    fh = jnp.full((16,), img_h, jnp.float32)
    scale = jnp.concatenate([fw, fh])
    tl = logits[:, _QTAIL:, :].reshape(-1)
    tb = boxes[:, _QTAIL:, :].reshape(-1)
    det_t, tdet = _run(lg_t, bx_t, scale, tl, tb)
    det = jnp.transpose(det_t, (1, 2, 0))
    det = det.at[:, _QTAIL:, :].set(tdet.reshape(n, q - _QTAIL, 6))
    mask = det[..., 1] >= _CONF
    return det, mask


# speedup vs baseline: 10.3058x; 1.1546x over previous
import jax
import jax.numpy as jnp
from jax import lax
from jax.experimental import pallas as pl
from jax.experimental.pallas import tpu as pltpu
from jax.experimental.pallas import tpu_sc as plsc

_CONF = 0.3
_NCLS = 91
_NIMG = 64
_NQ = 900
_QTAIL = 896
_CCH = 13
_NCH = 7
_W = 256


def _body(lg, bx, sc, det,
          rb0, rb1, rb2, bb0, bb1, bb2, bb3,
          db0, db1, db2, db3, db4, db5, mb, ab, sbuf,
          sem0, sem1, sem2, bsem):
    rbs = (rb0, rb1, rb2)
    sems = (sem0, sem1, sem2)
    bbs = (bb0, bb1, bb2, bb3)
    dbs = (db0, db1, db2, db3, db4, db5)

    wid = lax.axis_index("c") * 16 + lax.axis_index("s")
    band = wid // 4
    wslot = wid % 4
    n0 = pl.multiple_of(band * 8, 8)
    q0 = pl.multiple_of(jnp.minimum(wslot * _W, _NQ - 4 - _W), 128)

    pltpu.sync_copy(sc, sbuf)
    wv = sbuf[pl.ds(0, 16)]
    hv = sbuf[pl.ds(16, 16)]

    def start(c):
        return pltpu.async_copy(
            lg.at[pl.ds(c * _CCH, _CCH), pl.ds(n0, 8), pl.ds(q0, _W)],
            rbs[c % 3], sems[c % 3])

    cps = {c: start(c) for c in range(3)}
    bcps = [
        pltpu.async_copy(
            bx.at[pl.ds(n0, 8), comp, pl.ds(q0, _W)], bbs[comp], bsem)
        for comp in range(4)
    ]

    for c in range(_NCH):
        cps[c].wait()
        qb = rbs[c % 3]
        if c == _NCH - 1:
            for cp in bcps:
                cp.wait()

        def ibody(i, _, c=c, qb=qb):
            ms, ams = [], []
            for g in range(16):
                off = pl.multiple_of(g * 16, 16)
                if c == 0:
                    m = qb[0, i, pl.ds(off, 16)]
                    am = jnp.zeros((16,), jnp.float32)
                else:
                    m = mb[i, pl.ds(off, 16)]
                    am = ab[i, pl.ds(off, 16)]
                ms.append(m)
                ams.append(am)
            for cl in range(1 if c == 0 else 0, _CCH):
                for g in range(16):
                    off = pl.multiple_of(g * 16, 16)
                    v = qb[cl, i, pl.ds(off, 16)]
                    gt = v > ms[g]
                    ms[g] = jnp.maximum(ms[g], v)
                    ams[g] = jnp.where(gt, float(c * _CCH + cl), ams[g])
            if c < _NCH - 1:
                for g in range(16):
                    off = pl.multiple_of(g * 16, 16)
                    mb[i, pl.ds(off, 16)] = ms[g]
                    ab[i, pl.ds(off, 16)] = ams[g]
            else:
                zero = jnp.zeros((16,), jnp.float32)
                for g in range(16):
                    off = pl.multiple_of(g * 16, 16)
                    m, am = ms[g], ams[g]
                    s = 1.0 / (1.0 + jnp.exp(-m))
                    keep = s >= _CONF
                    cx = bb0[i, pl.ds(off, 16)]
                    cy = bb1[i, pl.ds(off, 16)]
                    bw = bb2[i, pl.ds(off, 16)]
                    bh = bb3[i, pl.ds(off, 16)]
                    db0[i, pl.ds(off, 16)] = jnp.where(keep, am, zero)
                    db1[i, pl.ds(off, 16)] = jnp.where(keep, s, zero)
                    db2[i, pl.ds(off, 16)] = jnp.where(
                        keep, (cx - 0.5 * bw) * wv, zero)
                    db3[i, pl.ds(off, 16)] = jnp.where(
                        keep, (cy - 0.5 * bh) * hv, zero)
                    db4[i, pl.ds(off, 16)] = jnp.where(keep, bw * wv, zero)
                    db5[i, pl.ds(off, 16)] = jnp.where(keep, bh * hv, zero)
            return 0

        lax.fori_loop(0, 8, ibody, 0)
        if c + 3 < _NCH:
            cps[c + 3] = start(c + 3)

    for comp in range(6):
        pltpu.sync_copy(
            dbs[comp], det.at[comp, pl.ds(n0, 8), pl.ds(q0, _W)])


@jax.jit
def _run(lg_t, bx_t, scale):
    fn = pl.kernel(
        _body,
        out_type=jax.ShapeDtypeStruct((6, _NIMG, _NQ), jnp.float32),
        mesh=plsc.VectorSubcoreMesh(core_axis_name="c", subcore_axis_name="s"),
        scratch_types=(
            [pltpu.VMEM((_CCH, 8, _W), jnp.float32) for _ in range(3)]
            + [pltpu.VMEM((8, _W), jnp.float32) for _ in range(4)]
            + [pltpu.VMEM((8, _W), jnp.float32) for _ in range(6)]
            + [pltpu.VMEM((8, _W), jnp.float32) for _ in range(2)]
            + [pltpu.VMEM((32,), jnp.float32)]
            + [pltpu.SemaphoreType.DMA for _ in range(4)]
        ),
        compiler_params=pltpu.CompilerParams(
            needs_layout_passes=False, use_tc_tiling_on_sc=True),
    )
    return fn(lg_t, bx_t, scale)


def kernel(logits, boxes, img_h, img_w):
    n, q, c = logits.shape
    lg_t = jnp.transpose(logits, (2, 0, 1))
    bx_t = jnp.transpose(boxes, (0, 2, 1))
    fw = jnp.full((16,), img_w, jnp.float32)
    fh = jnp.full((16,), img_h, jnp.float32)
    scale = jnp.concatenate([fw, fh])
    det_t = _run(lg_t, bx_t, scale)
    det = jnp.transpose(det_t, (1, 2, 0))

    tlg = logits[:, _QTAIL:, :]
    tbx = boxes[:, _QTAIL:, :]
    tm = jnp.max(tlg, axis=-1)
    targ = jnp.argmax(tlg, axis=-1).astype(jnp.float32)
    ts = jax.nn.sigmoid(tm)
    sizes = jnp.stack([jnp.float32(img_w), jnp.float32(img_h),
                       jnp.float32(img_w), jnp.float32(img_h)])
    tb_xy = jnp.concatenate(
        [tbx[..., :2] - tbx[..., 2:] * 0.5, tbx[..., 2:]], axis=-1) * sizes
    tdet = jnp.concatenate([targ[..., None], ts[..., None], tb_xy], axis=-1)
    tdet = jnp.where((ts >= _CONF)[..., None], tdet, 0.0)
    det = det.at[:, _QTAIL:, :].set(tdet)
    mask = det[..., 1] >= _CONF
    return det, mask


# speedup vs baseline: 10.3714x; 1.0064x over previous
import jax
import jax.numpy as jnp
from jax import lax
from jax.experimental import pallas as pl
from jax.experimental.pallas import tpu as pltpu
from jax.experimental.pallas import tpu_sc as plsc

_CONF = 0.3
_NCLS = 91
_NIMG = 64
_NQ = 900
_QTAIL = 896
_CCH = 13
_NCH = 7
_W = 256


def _body(lg, bx, sc, det,
          rb0, rb1, rb2, bb,
          db0, db1, db2, db3, db4, db5, mb, ab, sbuf,
          sem0, sem1, sem2, bsem):
    rbs = (rb0, rb1, rb2)
    sems = (sem0, sem1, sem2)
    dbs = (db0, db1, db2, db3, db4, db5)

    wid = lax.axis_index("c") * 16 + lax.axis_index("s")
    band = wid // 4
    wslot = wid % 4
    n0 = pl.multiple_of(band * 8, 8)
    q0 = pl.multiple_of(jnp.minimum(wslot * _W, _NQ - 4 - _W), 128)

    pltpu.sync_copy(sc, sbuf)
    wv = sbuf[pl.ds(0, 16)]
    hv = sbuf[pl.ds(16, 16)]

    def start(c):
        return pltpu.async_copy(
            lg.at[pl.ds(c * _CCH, _CCH), pl.ds(n0, 8), pl.ds(q0, _W)],
            rbs[c % 3], sems[c % 3])

    cps = {c: start(c) for c in range(3)}
    bcp = pltpu.async_copy(
        bx.at[pl.ds(n0, 8), :, pl.ds(q0, _W)], bb, bsem)

    for c in range(_NCH):
        cps[c].wait()
        qb = rbs[c % 3]
        if c == _NCH - 1:
            bcp.wait()

        def ibody(i, _, c=c, qb=qb):
            ms, ams = [], []
            for g in range(16):
                off = pl.multiple_of(g * 16, 16)
                if c == 0:
                    m = qb[0, i, pl.ds(off, 16)]
                    am = jnp.zeros((16,), jnp.float32)
                else:
                    m = mb[i, pl.ds(off, 16)]
                    am = ab[i, pl.ds(off, 16)]
                ms.append(m)
                ams.append(am)
            for cl in range(1 if c == 0 else 0, _CCH):
                for g in range(16):
                    off = pl.multiple_of(g * 16, 16)
                    v = qb[cl, i, pl.ds(off, 16)]
                    gt = v > ms[g]
                    ms[g] = jnp.maximum(ms[g], v)
                    ams[g] = jnp.where(gt, float(c * _CCH + cl), ams[g])
            if c < _NCH - 1:
                for g in range(16):
                    off = pl.multiple_of(g * 16, 16)
                    mb[i, pl.ds(off, 16)] = ms[g]
                    ab[i, pl.ds(off, 16)] = ams[g]
            else:
                zero = jnp.zeros((16,), jnp.float32)
                for g in range(16):
                    off = pl.multiple_of(g * 16, 16)
                    m, am = ms[g], ams[g]
                    s = 1.0 / (1.0 + jnp.exp(-m))
                    keep = s >= _CONF
                    cx = bb[i, 0, pl.ds(off, 16)]
                    cy = bb[i, 1, pl.ds(off, 16)]
                    bw = bb[i, 2, pl.ds(off, 16)]
                    bh = bb[i, 3, pl.ds(off, 16)]
                    db0[i, pl.ds(off, 16)] = jnp.where(keep, am, zero)
                    db1[i, pl.ds(off, 16)] = jnp.where(keep, s, zero)
                    db2[i, pl.ds(off, 16)] = jnp.where(
                        keep, (cx - 0.5 * bw) * wv, zero)
                    db3[i, pl.ds(off, 16)] = jnp.where(
                        keep, (cy - 0.5 * bh) * hv, zero)
                    db4[i, pl.ds(off, 16)] = jnp.where(keep, bw * wv, zero)
                    db5[i, pl.ds(off, 16)] = jnp.where(keep, bh * hv, zero)
            return 0

        lax.fori_loop(0, 8, ibody, 0)
        if c + 3 < _NCH:
            cps[c + 3] = start(c + 3)

    for comp in range(6):
        pltpu.sync_copy(
            dbs[comp], det.at[comp, pl.ds(n0, 8), pl.ds(q0, _W)])


@jax.jit
def _run(lg_t, bx_t, scale):
    fn = pl.kernel(
        _body,
        out_type=jax.ShapeDtypeStruct((6, _NIMG, _NQ), jnp.float32),
        mesh=plsc.VectorSubcoreMesh(core_axis_name="c", subcore_axis_name="s"),
        scratch_types=(
            [pltpu.VMEM((_CCH, 8, _W), jnp.float32) for _ in range(3)]
            + [pltpu.VMEM((8, 4, _W), jnp.float32)]
            + [pltpu.VMEM((8, _W), jnp.float32) for _ in range(6)]
            + [pltpu.VMEM((8, _W), jnp.float32) for _ in range(2)]
            + [pltpu.VMEM((32,), jnp.float32)]
            + [pltpu.SemaphoreType.DMA for _ in range(4)]
        ),
        compiler_params=pltpu.CompilerParams(
            needs_layout_passes=False, use_tc_tiling_on_sc=True),
    )
    return fn(lg_t, bx_t, scale)


def kernel(logits, boxes, img_h, img_w):
    n, q, c = logits.shape
    lg_t = jnp.transpose(logits, (2, 0, 1))
    bx_t = jnp.transpose(boxes, (0, 2, 1))
    fw = jnp.full((16,), img_w, jnp.float32)
    fh = jnp.full((16,), img_h, jnp.float32)
    scale = jnp.concatenate([fw, fh])
    det_t = _run(lg_t, bx_t, scale)
    det = jnp.transpose(det_t, (1, 2, 0))

    tlg = logits[:, _QTAIL:, :]
    tbx = boxes[:, _QTAIL:, :]
    tm = jnp.max(tlg, axis=-1)
    targ = jnp.argmax(tlg, axis=-1).astype(jnp.float32)
    ts = jax.nn.sigmoid(tm)
    sizes = jnp.stack([jnp.float32(img_w), jnp.float32(img_h),
                       jnp.float32(img_w), jnp.float32(img_h)])
    tb_xy = jnp.concatenate(
        [tbx[..., :2] - tbx[..., 2:] * 0.5, tbx[..., 2:]], axis=-1) * sizes
    tdet = jnp.concatenate([targ[..., None], ts[..., None], tb_xy], axis=-1)
    tdet = jnp.where((ts >= _CONF)[..., None], tdet, 0.0)
    det = det.at[:, _QTAIL:, :].set(tdet)
    mask = det[..., 1] >= _CONF
    return det, mask
